# baseline (speedup 1.0000x reference)
import jax
import jax.numpy as jnp
from jax import lax
from jax.experimental import pallas as pl
from jax.experimental.pallas import tpu as pltpu
from jax.experimental.pallas import tpu_sc as plsc

NC = 2
NS = 16
NW = NC * NS
LANES = 16

BATCH = 16384
OUT_DIM = 32
ROWS_PER_BLOCK = 128 // OUT_DIM
B_PER_W = BATCH // NW
CHUNK = 128
N_CHUNKS = B_PER_W // CHUNK
NBUF = 2


def _sc_kernel(i_hbm, j_hbm, u_hbm, v_hbm, out_hbm,
               raw_u, raw_v, blk_u, blk_v, u_buf, v_buf, out_v, sem):
    wid = lax.axis_index("s") * NC + lax.axis_index("c")
    base = wid * B_PER_W

    pltpu.sync_copy(i_hbm.at[wid], raw_u)
    pltpu.sync_copy(j_hbm.at[wid], raw_v)
    for k in range(N_CHUNKS):
        for t in range(CHUNK // LANES):
            s = pl.ds(t * LANES, LANES)
            blk_u[k, s] = raw_u[k, s] >> 2
            blk_v[k, s] = raw_v[k, s] >> 2

    def fire(k, buf):
        cu = pltpu.async_copy(u_hbm.at[blk_u.at[k]], u_buf.at[buf], sem)
        cv = pltpu.async_copy(v_hbm.at[blk_v.at[k]], v_buf.at[buf], sem)
        return cu, cv

    def compute(k, buf):
        def gbody(g, _):
            s = pl.ds(g * LANES, LANES)
            cu0 = (raw_u[k, s] & 3) << 5
            cv0 = (raw_v[k, s] & 3) << 5
            rows = g * LANES + lax.iota(jnp.int32, LANES)
            acc = jnp.zeros((LANES,), jnp.float32)
            for d in range(OUT_DIM):
                ud = plsc.load_gather(u_buf.at[buf], [rows, cu0 + d])
                vd = plsc.load_gather(v_buf.at[buf], [rows, cv0 + d])
                acc = acc + ud * vd
            out_v[pl.ds(k * CHUNK + g * LANES, LANES)] = acc
            return 0

        lax.fori_loop(0, CHUNK // LANES, gbody, 0)

    copies = [None] * N_CHUNKS
    copies[0] = fire(0, 0)
    for k in range(N_CHUNKS):
        if k + 1 < N_CHUNKS:
            copies[k + 1] = fire(k + 1, (k + 1) % NBUF)
        cu, cv = copies[k]
        cu.wait()
        cv.wait()
        compute(k, k % NBUF)

    pltpu.sync_copy(out_v, out_hbm.at[pl.ds(base, B_PER_W)])


@jax.jit
def _run(i3, j3, u2, v2):
    mesh = plsc.VectorSubcoreMesh(
        core_axis_name="c", subcore_axis_name="s",
        num_cores=NC, num_subcores=NS)
    f = pl.kernel(
        _sc_kernel,
        out_type=jax.ShapeDtypeStruct((BATCH,), jnp.float32),
        mesh=mesh,
        compiler_params=pltpu.CompilerParams(needs_layout_passes=False),
        scratch_types=[
            pltpu.VMEM((N_CHUNKS, CHUNK), jnp.int32),
            pltpu.VMEM((N_CHUNKS, CHUNK), jnp.int32),
            pltpu.VMEM((N_CHUNKS, CHUNK), jnp.int32),
            pltpu.VMEM((N_CHUNKS, CHUNK), jnp.int32),
            pltpu.VMEM((NBUF, CHUNK, 128), jnp.float32),
            pltpu.VMEM((NBUF, CHUNK, 128), jnp.float32),
            pltpu.VMEM((B_PER_W,), jnp.float32),
            pltpu.SemaphoreType.DMA,
        ],
    )
    return f(i3, j3, u2, v2)


def kernel(i, j, u_emb, v_emb):
    i3 = i.astype(jnp.int32).reshape(NW, N_CHUNKS, CHUNK)
    j3 = j.astype(jnp.int32).reshape(NW, N_CHUNKS, CHUNK)
    u2 = u_emb.reshape(-1, 128)
    v2 = v_emb.reshape(-1, 128)
    return _run(i3, j3, u2, v2)
